# Initial kernel scaffold; baseline (speedup 1.0000x reference)
import jax
import jax.numpy as jnp
from jax.experimental import pallas as pl


def kernel(input_word, input_pos1, input_pos2, word_table, pos1_table, pos2_table):
    raise NotImplementedError("write your pallas kernel here")


# speedup vs baseline: 5.2082x; 5.2082x over previous
import jax
import jax.numpy as jnp
from jax import lax
from jax.experimental import pallas as pl
from jax.experimental.pallas import tpu as pltpu
from jax.experimental.pallas import tpu_sc as plsc

B, S = 4096, 200
WORD_DIM = 128
POS_ROWS = 201
POS_SIZE = 16
OUT_DIM = WORD_DIM + 2 * POS_SIZE

NC, NS = 2, 16
NW = NC * NS
N = B * S
PER_W = N // NW
CHUNK = 256
NITER = PER_W // CHUNK
WSUB = CHUNK // 128


def _emb_kernel(widx_hbm, p1idx_hbm, p2idx_hbm, wtab_hbm, p1tab_hbm, p2tab_hbm,
                out_hbm, widx_v, p1idx_v, p2idx_v, outbuf_v, p1tab_v, p2tab_v,
                sem_w):
    wid = lax.axis_index("s") * NC + lax.axis_index("c")
    base = wid * PER_W

    pltpu.sync_copy(p1tab_hbm, p1tab_v)
    pltpu.sync_copy(p2tab_hbm, p2tab_v)

    lane = lax.iota(jnp.int32, 16)

    @pl.loop(0, NITER)
    def _(it):
        off = base + it * CHUNK
        pltpu.sync_copy(widx_hbm.at[pl.ds(off, CHUNK)], widx_v)
        pltpu.sync_copy(p1idx_hbm.at[pl.ds(off, CHUNK)], p1idx_v)
        pltpu.sync_copy(p2idx_hbm.at[pl.ds(off, CHUNK)], p2idx_v)

        copies = [
            pltpu.async_copy(
                wtab_hbm.at[widx_v.at[pl.ds(h * 128, 128)]],
                outbuf_v.at[pl.ds(h * 128, 128), pl.ds(0, WORD_DIM)],
                sem_w)
            for h in range(WSUB)
        ]

        @pl.loop(0, CHUNK // 16)
        def _(g):
            rowv = g * 16 + lane
            pv1 = p1idx_v[pl.ds(g * 16, 16)] * POS_SIZE
            pv2 = p2idx_v[pl.ds(g * 16, 16)] * POS_SIZE
            for c in range(POS_SIZE):
                v1 = plsc.load_gather(p1tab_v, [pv1 + c])
                plsc.store_scatter(
                    outbuf_v, [rowv, jnp.full((16,), WORD_DIM + c, jnp.int32)],
                    v1)
                v2 = plsc.load_gather(p2tab_v, [pv2 + c])
                plsc.store_scatter(
                    outbuf_v,
                    [rowv, jnp.full((16,), WORD_DIM + POS_SIZE + c, jnp.int32)],
                    v2)

        for cp in copies:
            cp.wait()
        pltpu.sync_copy(outbuf_v, out_hbm.at[pl.ds(off, CHUNK)])


@jax.jit
def _run(widx2d, p1idx, p2idx, word_table, pos1_flat, pos2_flat):
    mesh = plsc.VectorSubcoreMesh(core_axis_name="c", subcore_axis_name="s",
                                  num_cores=NC, num_subcores=NS)
    return pl.kernel(
        _emb_kernel,
        out_type=jax.ShapeDtypeStruct((N, OUT_DIM), jnp.float32),
        mesh=mesh,
        compiler_params=pltpu.CompilerParams(needs_layout_passes=False),
        scratch_types=[
            pltpu.VMEM((CHUNK,), jnp.int32),
            pltpu.VMEM((CHUNK,), jnp.int32),
            pltpu.VMEM((CHUNK,), jnp.int32),
            pltpu.VMEM((CHUNK, OUT_DIM), jnp.float32),
            pltpu.VMEM((POS_ROWS * POS_SIZE,), jnp.float32),
            pltpu.VMEM((POS_ROWS * POS_SIZE,), jnp.float32),
            pltpu.SemaphoreType.DMA,
        ],
    )(widx2d, p1idx, p2idx, word_table, pos1_flat, pos2_flat)


def kernel(input_word, input_pos1, input_pos2, word_table, pos1_table, pos2_table):
    widx2d = input_word.reshape(-1).astype(jnp.int32)
    p1idx = input_pos1.reshape(-1).astype(jnp.int32)
    p2idx = input_pos2.reshape(-1).astype(jnp.int32)
    out = _run(widx2d, p1idx, p2idx, word_table,
               pos1_table.reshape(-1), pos2_table.reshape(-1))
    return out.reshape(B, S, OUT_DIM)
